# baseline (speedup 1.0000x reference)
import functools

import jax
import jax.numpy as jnp
from jax import lax
from jax.experimental import pallas as pl
from jax.experimental.pallas import tpu as pltpu
from jax.experimental.pallas import tpu_sc as plsc

H = 128
W = 128
C = 128
N = H * W

_TAPS = [(dy, dx) for dx in (-1, 0, 1) for dy in (-1, 0, 1)]


def _shift_rows(a, o):
    if o == 0:
        return a
    z = jnp.zeros((abs(o), a.shape[1]), a.dtype)
    if o > 0:
        return jnp.concatenate([a[o:], z], axis=0)
    return jnp.concatenate([z, a[:o]], axis=0)


def _col_iota():
    return lax.broadcasted_iota(jnp.int32, (N, 1), 0) % W


def _dx_shift(x, col, dx):
    if dx == 0:
        return x
    if dx == 1:
        return jnp.where(col != W - 1, _shift_rows(x, 1), 0.0)
    return jnp.where(col != 0, _shift_rows(x, -1), 0.0)


def _acc_tap(acc_ref, t, dy, first):
    if first:
        acc_ref[...] = _shift_rows(t, dy * W)
    elif dy == 0:
        acc_ref[...] = acc_ref[...] + t
    elif dy == 1:
        acc_ref[0:N - W, :] = acc_ref[0:N - W, :] + t[W:N, :]
    else:
        acc_ref[W:N, :] = acc_ref[W:N, :] + t[0:N - W, :]


def _conv_taps(x, col, w_ref, acc_ref, wsel):
    for k, (dy, dx) in enumerate(_TAPS):
        xdx = _dx_shift(x, col, dx)
        t = jnp.dot(xdx, w_ref[wsel(dy + 1, dx + 1)],
                    preferred_element_type=jnp.float32)
        _acc_tap(acc_ref, t, dy, k == 0)


def _conv_bn_body(x_ref, wt_ref, g_ref, be_ref, out_ref):
    _conv_taps(x_ref[...], _col_iota(), wt_ref, out_ref,
               lambda ky, kx: ky * 3 + kx)
    acc = out_ref[...]
    m = jnp.mean(acc, axis=0, keepdims=True)
    v = jnp.mean((acc - m) ** 2, axis=0, keepdims=True)
    pre = g_ref[...] * ((acc - m) * lax.rsqrt(v + 1e-5)) + be_ref[...]
    out_ref[...] = jnp.maximum(pre, 0.0)


def _pools_body(pre_ref, tl_ref, br_ref):
    pre = pre_ref[...]
    col = _col_iota()
    top = pre
    for s in range(7):
        top = jnp.maximum(top, _shift_rows(top, W << s))
    tl_ref[...] = top
    left = pre
    for s in range(7):
        d = 1 << s
        left = jnp.maximum(left, jnp.where(col < W - d, _shift_rows(left, d), 0.0))
    tl_ref[...] = tl_ref[...] + left

    bot = pre
    for s in range(7):
        bot = jnp.maximum(bot, _shift_rows(bot, -(W << s)))
    br_ref[...] = bot
    right = pre
    for s in range(7):
        d = 1 << s
        right = jnp.maximum(right, jnp.where(col >= d, _shift_rows(right, -d), 0.0))
    br_ref[...] = br_ref[...] + right


def _pre_pools(x2, wt, g, be):
    pre = pl.pallas_call(
        _conv_bn_body,
        out_shape=jax.ShapeDtypeStruct((N, C), jnp.float32),
    )(x2, wt, g, be)
    return pl.pallas_call(
        _pools_body,
        out_shape=(
            jax.ShapeDtypeStruct((N, C), jnp.float32),
            jax.ShapeDtypeStruct((N, C), jnp.float32),
        ),
    )(pre)


def _heads_body(s_ref, wt1_ref, b1_ref, w2_ref, b2_ref, out_ref, acc_ref):
    _conv_taps(s_ref[...], _col_iota(), wt1_ref, acc_ref,
               lambda ky, kx: (ky * 3 + kx, 0))
    hh = jnp.maximum(acc_ref[...] + b1_ref[0], 0.0)
    t2 = jnp.dot(hh, w2_ref[0], preferred_element_type=jnp.float32)

    j = pl.program_id(0)

    @pl.when(j == 0)
    def _():
        out_ref[...] = t2 + b2_ref[...]

    @pl.when(j != 0)
    def _():
        out_ref[...] = out_ref[...] + t2


_HEADS_ORDER = ('tl_heats', 'tl_tag', 'tl_regr', 'br_heats', 'br_tag', 'br_regr')
_COL0 = {'heats': 3, 'tag': 0, 'regr': 1}


def _pack_weights(params):
    wcat = jnp.stack([params['w_pre']] + [params[n]['w1'] for n in _HEADS_ORDER])
    wt_all = jnp.transpose(wcat, (3, 4, 0, 2, 1)).reshape(9, 7, C, C)
    b1 = jnp.stack([params[n]['b1'] for n in _HEADS_ORDER]).reshape(6, 1, C)
    w2s, b2s = [], []
    for n in _HEADS_ORDER:
        c0 = _COL0[n.split('_')[1]]
        w2v = params[n]['w2'][:, :, 0, 0]
        c_out = w2v.shape[0]
        w2s.append(jnp.pad(jnp.transpose(w2v), ((0, 0), (c0, 128 - c0 - c_out))))
        b2s.append(jnp.pad(params[n]['b2'], (c0, 128 - c0 - c_out)))
    w2 = jnp.stack(w2s)
    b2 = [(b2s[0] + b2s[1] + b2s[2]).reshape(1, 128),
          (b2s[3] + b2s[4] + b2s[5]).reshape(1, 128)]
    return wt_all, b1, w2, b2


def _heads_side(smap, w1h, b1, w2, b2):
    return pl.pallas_call(
        _heads_body,
        grid=(3,),
        in_specs=[
            pl.BlockSpec((N, C), lambda j: (0, 0)),
            pl.BlockSpec((9, 1, C, C), lambda j: (0, j, 0, 0)),
            pl.BlockSpec((1, 1, C), lambda j: (j, 0, 0)),
            pl.BlockSpec((1, C, 128), lambda j: (j, 0, 0)),
            pl.BlockSpec((1, 128), lambda j: (0, 0)),
        ],
        out_specs=pl.BlockSpec((N, 128), lambda j: (0, 0)),
        out_shape=jax.ShapeDtypeStruct((N, 128), jnp.float32),
        scratch_shapes=[pltpu.VMEM((N, C), jnp.float32)],
    )(smap, w1h, b1, w2, b2)


def _gather_sc(tl_table, br_table, tl_idx, br_idx):
    info = plsc.get_sparse_core_info()
    nw = info.num_cores * info.num_subcores
    m = tl_idx.shape[0]
    per_side = nw // 2
    b_per_w = m // per_side
    mesh = plsc.VectorSubcoreMesh(core_axis_name="c", subcore_axis_name="s")

    @functools.partial(
        pl.kernel, mesh=mesh,
        out_type=jax.ShapeDtypeStruct((2 * m, 128), jnp.float32),
        scratch_types=[
            pltpu.VMEM((b_per_w,), jnp.int32),
            pltpu.VMEM((b_per_w, 128), jnp.float32),
            pltpu.SemaphoreType.DMA,
        ],
    )
    def k(tl_t, br_t, tl_i, br_i, out_hbm, idx_v, rows_v, sem):
        wid = lax.axis_index("s") * info.num_cores + lax.axis_index("c")

        @pl.when(wid < per_side)
        def _():
            base = wid * b_per_w
            pltpu.sync_copy(tl_i.at[pl.ds(base, b_per_w)], idx_v)
            pltpu.async_copy(tl_t.at[idx_v], rows_v, sem).wait()
            pltpu.sync_copy(rows_v, out_hbm.at[pl.ds(base, b_per_w)])

        @pl.when(wid >= per_side)
        def _():
            base = (wid - per_side) * b_per_w
            pltpu.sync_copy(br_i.at[pl.ds(base, b_per_w)], idx_v)
            pltpu.async_copy(br_t.at[idx_v], rows_v, sem).wait()
            pltpu.sync_copy(rows_v, out_hbm.at[pl.ds(m + base, b_per_w)])

    return k(tl_table, br_table, tl_idx, br_idx)


def kernel(x, tl_inds, br_inds, params):
    x2 = jnp.transpose(x.reshape(C, N))
    wt_all, b1, w2, b2 = _pack_weights(params)
    g = params['g_pre'].reshape(1, C)
    be = params['be_pre'].reshape(1, C)
    tl_map, br_map = _pre_pools(x2, wt_all[:, 0], g, be)
    tl_table = _heads_side(tl_map, wt_all[:, 1:4], b1[0:3], w2[0:3], b2[0])
    br_table = _heads_side(br_map, wt_all[:, 4:7], b1[3:6], w2[3:6], b2[1])

    gat = _gather_sc(tl_table, br_table,
                     tl_inds.reshape(-1).astype(jnp.int32),
                     br_inds.reshape(-1).astype(jnp.int32))

    m = tl_inds.shape[1]
    tl_heat = tl_table[:, 3].reshape(1, 1, H, W)
    br_heat = br_table[:, 3].reshape(1, 1, H, W)
    tl_tags = gat[0:m, 0:1][None]
    br_tags = gat[m:2 * m, 0:1][None]
    tl_regs = gat[0:m, 1:3][None]
    br_regs = gat[m:2 * m, 1:3][None]
    return (tl_heat, br_heat, tl_tags, br_tags, tl_regs, br_regs)
